# Initial kernel scaffold; baseline (speedup 1.0000x reference)
import jax
import jax.numpy as jnp
from jax.experimental import pallas as pl


def kernel(nfeat, edge_index, efeat, Wq, bq, Wk, bk, We, be, Wr, br):
    raise NotImplementedError("write your pallas kernel here")


# speedup vs baseline: 2.9198x; 2.9198x over previous
import functools

import jax
import jax.numpy as jnp
from jax import lax
from jax.experimental import pallas as pl
from jax.experimental.pallas import tpu as pltpu
from jax.experimental.pallas import tpu_sc as plsc

_N = 10000
_E = 320000
_D = 128
_DE = 16
_H = 128
_NP = 10240
_NTILES = 32
_EPW = _E // _NTILES
_CH = 80
_NCH = _EPW // _CH
_RPT = _NP // 16
_BN = 1024


def _node_proj_body(x_ref, wq_ref, bq_ref, wk_ref, bk_ref, eq_ref, ek_ref):
    x = x_ref[...]
    dn = (((1,), (1,)), ((), ()))
    eq_ref[...] = lax.dot_general(x, wq_ref[...], dn,
                                  preferred_element_type=jnp.float32) + bq_ref[...]
    ek_ref[...] = lax.dot_general(x, wk_ref[...], dn,
                                  preferred_element_type=jnp.float32) + bk_ref[...]


def _node_proj(x, wq, bq2, wk, bk2):
    return pl.pallas_call(
        _node_proj_body,
        grid=(pl.cdiv(_N, _BN),),
        in_specs=[
            pl.BlockSpec((_BN, _D), lambda i: (i, 0)),
            pl.BlockSpec((_H, _D), lambda i: (0, 0)),
            pl.BlockSpec((1, _H), lambda i: (0, 0)),
            pl.BlockSpec((_H, _D), lambda i: (0, 0)),
            pl.BlockSpec((1, _H), lambda i: (0, 0)),
        ],
        out_specs=[
            pl.BlockSpec((_BN, _H), lambda i: (i, 0)),
            pl.BlockSpec((_BN, _H), lambda i: (i, 0)),
        ],
        out_shape=[
            jax.ShapeDtypeStruct((_N, _H), jnp.float32),
            jax.ShapeDtypeStruct((_N, _H), jnp.float32),
        ],
    )(x, wq, bq2, wk, bk2)


def _edge_proj_body(ef_ref, we_ref, be_ref, e_ref):
    e_ref[...] = lax.dot_general(
        ef_ref[...], we_ref[...], (((1,), (1,)), ((), ())),
        preferred_element_type=jnp.float32) + be_ref[...]


def _edge_proj(efeat, we, be2):
    be_blk = 1280
    return pl.pallas_call(
        _edge_proj_body,
        grid=(_E // be_blk,),
        in_specs=[
            pl.BlockSpec((be_blk, _DE), lambda i: (i, 0)),
            pl.BlockSpec((_H, _DE), lambda i: (0, 0)),
            pl.BlockSpec((1, _H), lambda i: (0, 0)),
        ],
        out_specs=pl.BlockSpec((be_blk, _H), lambda i: (i, 0)),
        out_shape=jax.ShapeDtypeStruct((_E, _H), jnp.float32),
    )(efeat, we, be2)


def _out_proj_body(a_ref, b_ref, wr_ref, br_ref, o_ref):
    acc = a_ref[...] + b_ref[...]
    o_ref[...] = lax.dot_general(
        acc, wr_ref[...], (((1,), (1,)), ((), ())),
        preferred_element_type=jnp.float32) + br_ref[...]


def _out_proj(ftp, wr, br2):
    nb = _NP // _BN
    return pl.pallas_call(
        _out_proj_body,
        grid=(nb,),
        in_specs=[
            pl.BlockSpec((_BN, _H), lambda i: (i, 0)),
            pl.BlockSpec((_BN, _H), lambda i, nb=nb: (i + nb, 0)),
            pl.BlockSpec((_H, _H), lambda i: (0, 0)),
            pl.BlockSpec((1, _H), lambda i: (0, 0)),
        ],
        out_specs=pl.BlockSpec((_BN, _H), lambda i: (i, 0)),
        out_shape=jax.ShapeDtypeStruct((_NP, _H), jnp.float32),
    )(ftp, ftp, wr, br2)


def _sc_edge_body(eq_hbm, ek_hbm, e_hbm, src_hbm, dst_hbm, out_hbm,
                  srcv, dstv, ebuf, qbuf, kbuf, ft_sh, sem_e, sem_q, sem_k):
    c = lax.axis_index("c")
    s = lax.axis_index("s")
    wid = c * 16 + s

    def _zero_row(r, carry):
        for j in range(8):
            ebuf[r, pl.ds(j * 16, 16)] = jnp.zeros((16,), jnp.float32)
        return carry

    lax.fori_loop(0, _CH, _zero_row, 0)
    for t in range(_RPT // _CH):
        pltpu.sync_copy(ebuf, ft_sh.at[pl.ds(s * _RPT + t * _CH, _CH)])
    plsc.subcore_barrier()

    def _chunk(i, carry):
        base = wid * _EPW + i * _CH
        pltpu.sync_copy(src_hbm.at[pl.ds(base, _CH)], srcv)
        pltpu.sync_copy(dst_hbm.at[pl.ds(base, _CH)], dstv)
        ce = pltpu.async_copy(e_hbm.at[pl.ds(base, _CH)], ebuf, sem_e)
        cq = pltpu.async_copy(eq_hbm.at[dstv], qbuf, sem_q)
        ck = pltpu.async_copy(ek_hbm.at[srcv], kbuf, sem_k)
        ce.wait()
        cq.wait()
        ck.wait()

        def _row(r, rc):
            for j in range(8):
                sl = pl.ds(j * 16, 16)
                v = ebuf[r, sl] + qbuf[r, sl] + kbuf[r, sl]
                ebuf[r, sl] = jnp.maximum(v, 0.0)
            return rc

        lax.fori_loop(0, _CH, _row, 0)
        pltpu.sync_copy(ebuf, ft_sh.at[dstv], add=True)
        return carry

    lax.fori_loop(0, _NCH, _chunk, 0)
    plsc.subcore_barrier()

    pltpu.sync_copy(ft_sh.at[pl.ds(s * _RPT, _RPT)],
                    out_hbm.at[pl.ds(c * _NP + s * _RPT, _RPT)])


@functools.lru_cache(maxsize=1)
def _sc_edge_kernel():
    return functools.partial(
        pl.kernel,
        out_type=jax.ShapeDtypeStruct((2 * _NP, _H), jnp.float32),
        mesh=plsc.VectorSubcoreMesh(core_axis_name="c", subcore_axis_name="s",
                                    num_cores=2, num_subcores=16),
        scratch_types=[
            pltpu.VMEM((_CH,), jnp.int32),
            pltpu.VMEM((_CH,), jnp.int32),
            pltpu.VMEM((_CH, _H), jnp.float32),
            pltpu.VMEM((_CH, _H), jnp.float32),
            pltpu.VMEM((_CH, _H), jnp.float32),
            pltpu.VMEM_SHARED((_NP, _H), jnp.float32),
            pltpu.SemaphoreType.DMA,
            pltpu.SemaphoreType.DMA,
            pltpu.SemaphoreType.DMA,
        ],
    )(_sc_edge_body)


def kernel(nfeat, edge_index, efeat, Wq, bq, Wk, bk, We, be, Wr, br):
    src = edge_index[0]
    dst = edge_index[1]
    eq, ek = _node_proj(nfeat, Wq, bq.reshape(1, _H), Wk, bk.reshape(1, _H))
    e = _edge_proj(efeat, We, be.reshape(1, _H))
    ftp = _sc_edge_kernel()(eq, ek, e, src, dst)
    rst = _out_proj(ftp, Wr, br.reshape(1, _H))
    return rst[:_N]
